# baseline (speedup 1.0000x reference)
import functools

import jax
import jax.numpy as jnp
from jax import lax
from jax.experimental import pallas as pl
from jax.experimental.pallas import tpu as pltpu
from jax.experimental.pallas import tpu_sc as plsc

BETA = 0.5

N = 10000
NH = N // 2
E = 160000
EP = 163840
E_BLK = 2048
N_BLK = 1000
AGG_ROWS = 10112
ROWS_PER_TILE = 632
CHUNK = 128
CHUNKS_PER_TILE = EP // 16 // CHUNK
G_CHUNKS = EP // 32 // CHUNK
PK_ROWS_PER_TILE = 320
PK_LAST = NH - 15 * PK_ROWS_PER_TILE


def _unpack_pair(words, parity):
    sel = jnp.where(parity == 1, words[:, 64:], words[:, :64])
    even = lax.bitcast_convert_type(sel << 16, jnp.float32)
    odd = lax.bitcast_convert_type(
        sel & jnp.int32(-65536), jnp.float32)
    return even, odd


def _edge_mlp_body(xr, xc, pr, pc, l0, l1,
                   w1ae, w1ao, w1be, w1bo, b1c, w20, w21, b20, b21, out):
    re_, ro = _unpack_pair(xr[...], pr[...])
    ce, co = _unpack_pair(xc[...], pc[...])
    h = jnp.dot(re_, w1ae[...], preferred_element_type=jnp.float32)
    h += jnp.dot(ro, w1ao[...], preferred_element_type=jnp.float32)
    h += jnp.dot(ce, w1be[...], preferred_element_type=jnp.float32)
    h += jnp.dot(co, w1bo[...], preferred_element_type=jnp.float32)
    h = jnp.maximum(h + b1c[...], 0.0)
    m0 = jnp.dot(h[:, :256], w20[...], preferred_element_type=jnp.float32)
    m0 = jnp.maximum(m0 + b20[...], 0.0)
    m1 = jnp.dot(h[:, 256:], w21[...], preferred_element_type=jnp.float32)
    m1 = jnp.maximum(m1 + b21[...], 0.0)
    d = l0[...] - l1[...]
    p0 = 1.0 / (1.0 + jnp.exp(-BETA * d))
    msg = m0 * p0 + m1 * (1.0 - p0)
    out[0] = msg[:, :128]
    out[1] = msg[:, 128:]


def _edge_mlp(xr, xc, pr, pc, l0, l1, w1ae, w1ao, w1be, w1bo, b1c,
              w20, w21, b20, b21):
    e = xr.shape[0]
    grid = (e // E_BLK,)
    blk = lambda r, c: pl.BlockSpec((r, c), lambda i: (i, 0))
    full = lambda r, c: pl.BlockSpec((r, c), lambda i: (0, 0))
    return pl.pallas_call(
        _edge_mlp_body,
        grid=grid,
        in_specs=[
            blk(E_BLK, 128), blk(E_BLK, 128),
            blk(E_BLK, 1), blk(E_BLK, 1), blk(E_BLK, 1), blk(E_BLK, 1),
            full(64, 512), full(64, 512), full(64, 512), full(64, 512),
            full(1, 512),
            full(256, 256), full(256, 256), full(1, 256), full(1, 256),
        ],
        out_specs=pl.BlockSpec((2, E_BLK, 128), lambda i: (0, i, 0)),
        out_shape=jax.ShapeDtypeStruct((2, e, 128), jnp.float32),
    )(xr, xc, pr, pc, l0, l1, w1ae, w1ao, w1be, w1bo, b1c, w20, w21, b20, b21)


def _sc_gather_body(xpk_hbm, rowg_hbm, colg_hbm, xr_hbm, xc_hbm,
                    idx_r, idx_c, bufr, bufc, x_sp,
                    gsr0, gsr1, gsc0, gsc1, osr0, osr1, osc0, osc1):
    c = lax.axis_index("c")
    s = lax.axis_index("s")
    wid = s * 2 + c
    base = wid * (G_CHUNKS * CHUNK)
    @pl.when(s < 15)
    def _():
        pltpu.sync_copy(
            xpk_hbm.at[pl.ds(s * PK_ROWS_PER_TILE, PK_ROWS_PER_TILE), :],
            x_sp.at[pl.ds(s * PK_ROWS_PER_TILE, PK_ROWS_PER_TILE), :])

    @pl.when(s == 15)
    def _():
        pltpu.sync_copy(
            xpk_hbm.at[pl.ds(15 * PK_ROWS_PER_TILE, PK_LAST), :],
            x_sp.at[pl.ds(15 * PK_ROWS_PER_TILE, PK_LAST), :])

    pltpu.sync_copy(rowg_hbm.at[wid], idx_r)
    pltpu.sync_copy(colg_hbm.at[wid], idx_c)
    plsc.subcore_barrier()

    gsems_r = (gsr0, gsr1)
    gsems_c = (gsc0, gsc1)
    osems_r = (osr0, osr1)
    osems_c = (osc0, osc1)

    def out_r(j, p):
        return pltpu.make_async_copy(
            bufr.at[p], xr_hbm.at[pl.ds(base + j * CHUNK, CHUNK), :],
            osems_r[p])

    def out_c(j, p):
        return pltpu.make_async_copy(
            bufc.at[p], xc_hbm.at[pl.ds(base + j * CHUNK, CHUNK), :],
            osems_c[p])

    for p in (0, 1):
        pltpu.async_copy(x_sp.at[idx_r.at[p]], bufr.at[p], gsems_r[p])
        pltpu.async_copy(x_sp.at[idx_c.at[p]], bufc.at[p], gsems_c[p])

    def pair_body(g, carry):
        for p in (0, 1):
            j = 2 * g + p
            pltpu.make_async_copy(x_sp.at[idx_r.at[0]], bufr.at[p],
                                  gsems_r[p]).wait()
            out_r(j, p).start()
            pltpu.make_async_copy(x_sp.at[idx_c.at[0]], bufc.at[p],
                                  gsems_c[p]).wait()
            out_c(j, p).start()

            @pl.when(j + 2 < G_CHUNKS)
            def _():
                out_r(j, p).wait()
                pltpu.async_copy(x_sp.at[idx_r.at[j + 2]], bufr.at[p],
                                 gsems_r[p])
                out_c(j, p).wait()
                pltpu.async_copy(x_sp.at[idx_c.at[j + 2]], bufc.at[p],
                                 gsems_c[p])
        return carry

    lax.fori_loop(0, G_CHUNKS // 2, pair_body, 0, unroll=False)
    for p in (0, 1):
        out_r(0, p).wait()
        out_c(0, p).wait()


def _sc_gather(xpk, rowg, colg):
    mesh = plsc.VectorSubcoreMesh(core_axis_name="c", subcore_axis_name="s")
    f = pl.kernel(
        _sc_gather_body,
        out_type=(jax.ShapeDtypeStruct((EP, 128), jnp.int32),
                  jax.ShapeDtypeStruct((EP, 128), jnp.int32)),
        mesh=mesh,
        scratch_types=[
            pltpu.VMEM((G_CHUNKS, CHUNK), jnp.int32),
            pltpu.VMEM((G_CHUNKS, CHUNK), jnp.int32),
            pltpu.VMEM((2, CHUNK, 128), jnp.int32),
            pltpu.VMEM((2, CHUNK, 128), jnp.int32),
            pltpu.VMEM_SHARED((NH, 128), jnp.int32),
        ] + [pltpu.SemaphoreType.DMA] * 8,
    )
    return f(xpk, rowg, colg)


def _sc_scatter_body(msg_hbm, idx_hbm, zeros_hbm, out_hbm,
                     idx_v, buf, agg_sh, sem):
    c = lax.axis_index("c")
    s = lax.axis_index("s")
    pltpu.sync_copy(zeros_hbm.at[pl.ds(s * ROWS_PER_TILE, ROWS_PER_TILE), :],
                    agg_sh.at[pl.ds(s * ROWS_PER_TILE, ROWS_PER_TILE), :])
    pltpu.sync_copy(idx_hbm.at[s], idx_v)
    plsc.subcore_barrier()

    base = s * (CHUNKS_PER_TILE * CHUNK)

    def body(j, carry):
        pltpu.sync_copy(msg_hbm.at[c, pl.ds(base + j * CHUNK, CHUNK), :], buf)
        pltpu.sync_copy(buf, agg_sh.at[idx_v.at[j]], add=True)
        return carry

    lax.fori_loop(0, CHUNKS_PER_TILE, body, 0, unroll=False)
    plsc.subcore_barrier()

    @pl.when(s < 15)
    def _():
        pltpu.sync_copy(agg_sh.at[pl.ds(s * ROWS_PER_TILE, ROWS_PER_TILE), :],
                        out_hbm.at[c, pl.ds(s * ROWS_PER_TILE, ROWS_PER_TILE), :])

    @pl.when(s == 15)
    def _():
        pltpu.sync_copy(agg_sh.at[pl.ds(15 * ROWS_PER_TILE, N - 15 * ROWS_PER_TILE), :],
                        out_hbm.at[c, pl.ds(15 * ROWS_PER_TILE, N - 15 * ROWS_PER_TILE), :])


def _sc_scatter(msg2, idx3, zeros):
    mesh = plsc.VectorSubcoreMesh(core_axis_name="c", subcore_axis_name="s")
    f = pl.kernel(
        _sc_scatter_body,
        out_type=jax.ShapeDtypeStruct((2, N, 128), jnp.float32),
        mesh=mesh,
        scratch_types=[
            pltpu.VMEM((CHUNKS_PER_TILE, CHUNK), jnp.int32),
            pltpu.VMEM((CHUNK, 128), jnp.float32),
            pltpu.VMEM_SHARED((AGG_ROWS, 128), jnp.float32),
            pltpu.SemaphoreType.DMA,
        ],
    )
    return f(msg2, idx3, zeros)


def _node_mlp_body(x, agg0, agg1, wo1x, wo1a0, wo1a1, bo1, wo2, bo2, wo3, bo3, out):
    h1 = jnp.dot(x[...], wo1x[...], preferred_element_type=jnp.float32)
    h1 += jnp.dot(agg0[...], wo1a0[...], preferred_element_type=jnp.float32)
    h1 += jnp.dot(agg1[...], wo1a1[...], preferred_element_type=jnp.float32)
    h1 = jnp.maximum(h1 + bo1[...], 0.0)
    h2 = jnp.dot(h1, wo2[...], preferred_element_type=jnp.float32)
    h2 = jnp.maximum(h2 + bo2[...], 0.0)
    out[...] = x[...] + jnp.dot(h2, wo3[...], preferred_element_type=jnp.float32) + bo3[...]


def _node_mlp(x, agg0, agg1, wo1x, wo1a0, wo1a1, bo1, wo2, bo2, wo3, bo3):
    n = x.shape[0]
    grid = (n // N_BLK,)
    blk = lambda r, c: pl.BlockSpec((r, c), lambda i: (i, 0))
    full = lambda r, c: pl.BlockSpec((r, c), lambda i: (0, 0))
    return pl.pallas_call(
        _node_mlp_body,
        grid=grid,
        in_specs=[
            blk(N_BLK, 128), blk(N_BLK, 128), blk(N_BLK, 128),
            full(128, 256), full(128, 256), full(128, 256), full(1, 256),
            full(256, 256), full(1, 256),
            full(256, 128), full(1, 128),
        ],
        out_specs=blk(N_BLK, 128),
        out_shape=jax.ShapeDtypeStruct((n, 128), jnp.float32),
    )(x, agg0, agg1, wo1x, wo1a0, wo1a1, bo1, wo2, bo2, wo3, bo3)


def kernel(inputs, edge_index, logits, W1, b1, W2, b2, Wo1, bo1, Wo2, bo2, Wo3, bo3):
    row = edge_index[0]
    col = edge_index[1]

    w1a = jnp.concatenate([W1[0, :128, :], W1[1, :128, :]], axis=1)
    w1b = jnp.concatenate([W1[0, 128:, :], W1[1, 128:, :]], axis=1)
    w1ae, w1ao = w1a[0::2], w1a[1::2]
    w1be, w1bo = w1b[0::2], w1b[1::2]
    b1c = jnp.concatenate([b1[0], b1[1]])[None, :]
    w20, w21 = W2[0], W2[1]
    b20, b21 = b2[0][None, :], b2[1][None, :]
    wo1x = Wo1[:128, :]
    wo1a0, wo1a1 = Wo1[128:256, :], Wo1[256:, :]
    bo1r, bo2r, bo3r = bo1[None, :], bo2[None, :], bo3[None, :]

    pad = EP - E
    l0 = jnp.pad(logits[0], (0, pad))[:, None]
    l1 = jnp.pad(logits[1], (0, pad))[:, None]
    rp = jnp.pad(row, (0, pad))
    cp = jnp.pad(col, (0, pad))
    row_g = (rp >> 1).reshape(32, G_CHUNKS, CHUNK)
    col_g = (cp >> 1).reshape(32, G_CHUNKS, CHUNK)
    pr = (rp & 1)[:, None]
    pc = (cp & 1)[:, None]
    idx3 = jnp.pad(row, (0, pad), constant_values=N).reshape(16, CHUNKS_PER_TILE, CHUNK)
    zeros = jnp.zeros((AGG_ROWS, 128), jnp.float32)

    def step(x):
        xpk = lax.bitcast_convert_type(
            x.astype(jnp.bfloat16).reshape(NH, 128, 2), jnp.int32)
        xr, xc = _sc_gather(xpk, row_g, col_g)
        msg2 = _edge_mlp(xr, xc, pr, pc, l0, l1, w1ae, w1ao, w1be, w1bo,
                         b1c, w20, w21, b20, b21)
        agg2 = _sc_scatter(msg2, idx3, zeros)
        return _node_mlp(x, agg2[0], agg2[1], wo1x, wo1a0, wo1a1, bo1r,
                         Wo2, bo2r, Wo3, bo3r)

    x = inputs[0, :, :, 0]
    x1 = step(x)
    x2 = step(x1)
    return jnp.stack([x1, x2], axis=-1)[None]


# speedup vs baseline: 1.8287x; 1.8287x over previous
import functools

import jax
import jax.numpy as jnp
from jax import lax
from jax.experimental import pallas as pl
from jax.experimental.pallas import tpu as pltpu
from jax.experimental.pallas import tpu_sc as plsc

BETA = 0.5

N = 10000
NH = N // 2
E = 160000
EP = 163840
E_BLK = 2048
N_BLK = 1000
AGG_ROWS = 10112
ROWS_PER_TILE = 632
CHUNK = 128
CHUNKS_PER_TILE = EP // 16 // CHUNK
G_CHUNKS = EP // 32 // CHUNK
PK_ROWS_PER_TILE = 320
PK_LAST = NH - 15 * PK_ROWS_PER_TILE


def _unpack(words, parity):
    lo = lax.bitcast_convert_type(words << 16, jnp.float32)
    hi = lax.bitcast_convert_type(words & jnp.int32(-65536), jnp.float32)
    return jnp.where(parity == 1, hi, lo)


def _edge_mlp_body(xr, xc, pr, pc, l0, l1,
                   w1a, w1b, b1c, w20, w21, b20, b21, out):
    xrf = _unpack(xr[...], pr[...])
    xcf = _unpack(xc[...], pc[...])
    h = jnp.dot(xrf, w1a[...], preferred_element_type=jnp.float32)
    h += jnp.dot(xcf, w1b[...], preferred_element_type=jnp.float32)
    h = jnp.maximum(h + b1c[...], 0.0)
    m0 = jnp.dot(h[:, :256], w20[...], preferred_element_type=jnp.float32)
    m0 = jnp.maximum(m0 + b20[...], 0.0)
    m1 = jnp.dot(h[:, 256:], w21[...], preferred_element_type=jnp.float32)
    m1 = jnp.maximum(m1 + b21[...], 0.0)
    d = l0[...] - l1[...]
    p0 = 1.0 / (1.0 + jnp.exp(-BETA * d))
    msg = m0 * p0 + m1 * (1.0 - p0)
    out[0] = msg[:, :128]
    out[1] = msg[:, 128:]


def _edge_mlp(xr, xc, pr, pc, l0, l1, w1a, w1b, b1c, w20, w21, b20, b21):
    e = xr.shape[0]
    grid = (e // E_BLK,)
    blk = lambda r, c: pl.BlockSpec((r, c), lambda i: (i, 0))
    full = lambda r, c: pl.BlockSpec((r, c), lambda i: (0, 0))
    return pl.pallas_call(
        _edge_mlp_body,
        grid=grid,
        in_specs=[
            blk(E_BLK, 128), blk(E_BLK, 128),
            blk(E_BLK, 1), blk(E_BLK, 1), blk(E_BLK, 1), blk(E_BLK, 1),
            full(128, 512), full(128, 512), full(1, 512),
            full(256, 256), full(256, 256), full(1, 256), full(1, 256),
        ],
        out_specs=pl.BlockSpec((2, E_BLK, 128), lambda i: (0, i, 0)),
        out_shape=jax.ShapeDtypeStruct((2, e, 128), jnp.float32),
    )(xr, xc, pr, pc, l0, l1, w1a, w1b, b1c, w20, w21, b20, b21)


def _sc_gather_body(xpk_hbm, rowg_hbm, colg_hbm, xr_hbm, xc_hbm,
                    idx_r, idx_c, bufr, bufc, x_sp,
                    gsr0, gsr1, gsc0, gsc1, osr0, osr1, osc0, osc1):
    c = lax.axis_index("c")
    s = lax.axis_index("s")
    wid = s * 2 + c
    base = wid * (G_CHUNKS * CHUNK)
    @pl.when(s < 15)
    def _():
        pltpu.sync_copy(
            xpk_hbm.at[pl.ds(s * PK_ROWS_PER_TILE, PK_ROWS_PER_TILE), :],
            x_sp.at[pl.ds(s * PK_ROWS_PER_TILE, PK_ROWS_PER_TILE), :])

    @pl.when(s == 15)
    def _():
        pltpu.sync_copy(
            xpk_hbm.at[pl.ds(15 * PK_ROWS_PER_TILE, PK_LAST), :],
            x_sp.at[pl.ds(15 * PK_ROWS_PER_TILE, PK_LAST), :])

    pltpu.sync_copy(rowg_hbm.at[wid], idx_r)
    pltpu.sync_copy(colg_hbm.at[wid], idx_c)
    plsc.subcore_barrier()

    gsems_r = (gsr0, gsr1)
    gsems_c = (gsc0, gsc1)
    osems_r = (osr0, osr1)
    osems_c = (osc0, osc1)

    def out_r(j, p):
        return pltpu.make_async_copy(
            bufr.at[p], xr_hbm.at[pl.ds(base + j * CHUNK, CHUNK), :],
            osems_r[p])

    def out_c(j, p):
        return pltpu.make_async_copy(
            bufc.at[p], xc_hbm.at[pl.ds(base + j * CHUNK, CHUNK), :],
            osems_c[p])

    for p in (0, 1):
        pltpu.async_copy(x_sp.at[idx_r.at[p]], bufr.at[p], gsems_r[p])
        pltpu.async_copy(x_sp.at[idx_c.at[p]], bufc.at[p], gsems_c[p])

    def pair_body(g, carry):
        for p in (0, 1):
            j = 2 * g + p
            pltpu.make_async_copy(x_sp.at[idx_r.at[0]], bufr.at[p],
                                  gsems_r[p]).wait()
            out_r(j, p).start()
            pltpu.make_async_copy(x_sp.at[idx_c.at[0]], bufc.at[p],
                                  gsems_c[p]).wait()
            out_c(j, p).start()

            @pl.when(j + 2 < G_CHUNKS)
            def _():
                out_r(j, p).wait()
                pltpu.async_copy(x_sp.at[idx_r.at[j + 2]], bufr.at[p],
                                 gsems_r[p])
                out_c(j, p).wait()
                pltpu.async_copy(x_sp.at[idx_c.at[j + 2]], bufc.at[p],
                                 gsems_c[p])
        return carry

    lax.fori_loop(0, G_CHUNKS // 2, pair_body, 0, unroll=False)
    for p in (0, 1):
        out_r(0, p).wait()
        out_c(0, p).wait()


def _sc_gather(xpk, rowg, colg):
    mesh = plsc.VectorSubcoreMesh(core_axis_name="c", subcore_axis_name="s")
    f = pl.kernel(
        _sc_gather_body,
        out_type=(jax.ShapeDtypeStruct((EP, 128), jnp.int32),
                  jax.ShapeDtypeStruct((EP, 128), jnp.int32)),
        mesh=mesh,
        scratch_types=[
            pltpu.VMEM((G_CHUNKS, CHUNK), jnp.int32),
            pltpu.VMEM((G_CHUNKS, CHUNK), jnp.int32),
            pltpu.VMEM((2, CHUNK, 128), jnp.int32),
            pltpu.VMEM((2, CHUNK, 128), jnp.int32),
            pltpu.VMEM_SHARED((NH, 128), jnp.int32),
        ] + [pltpu.SemaphoreType.DMA] * 8,
    )
    return f(xpk, rowg, colg)


def _sc_scatter_body(msg_hbm, idx_hbm, zeros_hbm, out_hbm,
                     idx_v, buf, agg_sh, sem):
    c = lax.axis_index("c")
    s = lax.axis_index("s")
    pltpu.sync_copy(zeros_hbm.at[pl.ds(s * ROWS_PER_TILE, ROWS_PER_TILE), :],
                    agg_sh.at[pl.ds(s * ROWS_PER_TILE, ROWS_PER_TILE), :])
    pltpu.sync_copy(idx_hbm.at[s], idx_v)
    plsc.subcore_barrier()

    base = s * (CHUNKS_PER_TILE * CHUNK)

    def body(j, carry):
        pltpu.sync_copy(msg_hbm.at[c, pl.ds(base + j * CHUNK, CHUNK), :], buf)
        pltpu.sync_copy(buf, agg_sh.at[idx_v.at[j]], add=True)
        return carry

    lax.fori_loop(0, CHUNKS_PER_TILE, body, 0, unroll=False)
    plsc.subcore_barrier()

    @pl.when(s < 15)
    def _():
        pltpu.sync_copy(agg_sh.at[pl.ds(s * ROWS_PER_TILE, ROWS_PER_TILE), :],
                        out_hbm.at[c, pl.ds(s * ROWS_PER_TILE, ROWS_PER_TILE), :])

    @pl.when(s == 15)
    def _():
        pltpu.sync_copy(agg_sh.at[pl.ds(15 * ROWS_PER_TILE, N - 15 * ROWS_PER_TILE), :],
                        out_hbm.at[c, pl.ds(15 * ROWS_PER_TILE, N - 15 * ROWS_PER_TILE), :])


def _sc_scatter(msg2, idx3, zeros):
    mesh = plsc.VectorSubcoreMesh(core_axis_name="c", subcore_axis_name="s")
    f = pl.kernel(
        _sc_scatter_body,
        out_type=jax.ShapeDtypeStruct((2, N, 128), jnp.float32),
        mesh=mesh,
        scratch_types=[
            pltpu.VMEM((CHUNKS_PER_TILE, CHUNK), jnp.int32),
            pltpu.VMEM((CHUNK, 128), jnp.float32),
            pltpu.VMEM_SHARED((AGG_ROWS, 128), jnp.float32),
            pltpu.SemaphoreType.DMA,
        ],
    )
    return f(msg2, idx3, zeros)


def _node_mlp_body(x, agg0, agg1, wo1x, wo1a0, wo1a1, bo1, wo2, bo2, wo3, bo3, out):
    h1 = jnp.dot(x[...], wo1x[...], preferred_element_type=jnp.float32)
    h1 += jnp.dot(agg0[...], wo1a0[...], preferred_element_type=jnp.float32)
    h1 += jnp.dot(agg1[...], wo1a1[...], preferred_element_type=jnp.float32)
    h1 = jnp.maximum(h1 + bo1[...], 0.0)
    h2 = jnp.dot(h1, wo2[...], preferred_element_type=jnp.float32)
    h2 = jnp.maximum(h2 + bo2[...], 0.0)
    out[...] = x[...] + jnp.dot(h2, wo3[...], preferred_element_type=jnp.float32) + bo3[...]


def _node_mlp(x, agg0, agg1, wo1x, wo1a0, wo1a1, bo1, wo2, bo2, wo3, bo3):
    n = x.shape[0]
    grid = (n // N_BLK,)
    blk = lambda r, c: pl.BlockSpec((r, c), lambda i: (i, 0))
    full = lambda r, c: pl.BlockSpec((r, c), lambda i: (0, 0))
    return pl.pallas_call(
        _node_mlp_body,
        grid=grid,
        in_specs=[
            blk(N_BLK, 128), blk(N_BLK, 128), blk(N_BLK, 128),
            full(128, 256), full(128, 256), full(128, 256), full(1, 256),
            full(256, 256), full(1, 256),
            full(256, 128), full(1, 128),
        ],
        out_specs=blk(N_BLK, 128),
        out_shape=jax.ShapeDtypeStruct((n, 128), jnp.float32),
    )(x, agg0, agg1, wo1x, wo1a0, wo1a1, bo1, wo2, bo2, wo3, bo3)


def kernel(inputs, edge_index, logits, W1, b1, W2, b2, Wo1, bo1, Wo2, bo2, Wo3, bo3):
    row = edge_index[0]
    col = edge_index[1]

    w1a = jnp.concatenate([W1[0, :128, :], W1[1, :128, :]], axis=1)
    w1b = jnp.concatenate([W1[0, 128:, :], W1[1, 128:, :]], axis=1)
    b1c = jnp.concatenate([b1[0], b1[1]])[None, :]
    w20, w21 = W2[0], W2[1]
    b20, b21 = b2[0][None, :], b2[1][None, :]
    wo1x = Wo1[:128, :]
    wo1a0, wo1a1 = Wo1[128:256, :], Wo1[256:, :]
    bo1r, bo2r, bo3r = bo1[None, :], bo2[None, :], bo3[None, :]

    pad = EP - E
    l0 = jnp.pad(logits[0], (0, pad))[:, None]
    l1 = jnp.pad(logits[1], (0, pad))[:, None]
    rp = jnp.pad(row, (0, pad))
    cp = jnp.pad(col, (0, pad))
    prf = (rp >= NH).astype(jnp.int32)
    pcf = (cp >= NH).astype(jnp.int32)
    row_g = (rp - NH * prf).reshape(32, G_CHUNKS, CHUNK)
    col_g = (cp - NH * pcf).reshape(32, G_CHUNKS, CHUNK)
    pr = prf[:, None]
    pc = pcf[:, None]
    idx3 = jnp.pad(row, (0, pad), constant_values=N).reshape(16, CHUNKS_PER_TILE, CHUNK)
    zeros = jnp.zeros((AGG_ROWS, 128), jnp.float32)

    def step(x):
        lo = lax.bitcast_convert_type(
            x[:NH].astype(jnp.bfloat16), jnp.uint16).astype(jnp.int32)
        hi = lax.bitcast_convert_type(
            x[NH:].astype(jnp.bfloat16), jnp.uint16).astype(jnp.int32)
        xpk = jnp.bitwise_or(jnp.left_shift(hi, 16), lo)
        xr, xc = _sc_gather(xpk, row_g, col_g)
        msg2 = _edge_mlp(xr, xc, pr, pc, l0, l1, w1a, w1b,
                         b1c, w20, w21, b20, b21)
        agg2 = _sc_scatter(msg2, idx3, zeros)
        return _node_mlp(x, agg2[0], agg2[1], wo1x, wo1a0, wo1a1, bo1r,
                         Wo2, bo2r, Wo3, bo3r)

    x = inputs[0, :, :, 0]
    x1 = step(x)
    x2 = step(x1)
    return jnp.stack([x1, x2], axis=-1)[None]


# speedup vs baseline: 1.8535x; 1.0136x over previous
import functools

import jax
import jax.numpy as jnp
from jax import lax
from jax.experimental import pallas as pl
from jax.experimental.pallas import tpu as pltpu
from jax.experimental.pallas import tpu_sc as plsc

BETA = 0.5

N = 10000
NH = N // 2
E = 160000
EP = 163840
E_BLK = 2048
N_BLK = 1000
AGG_ROWS = 10112
ROWS_PER_TILE = 632
CHUNK = 128
CHUNKS_PER_TILE = EP // 16 // CHUNK
G_CHUNKS = EP // 32 // CHUNK
PK_ROWS_PER_TILE = 320
PK_LAST = NH - 15 * PK_ROWS_PER_TILE


def _unpack(words, parity):
    lo = lax.bitcast_convert_type(words << 16, jnp.float32)
    hi = lax.bitcast_convert_type(words & jnp.int32(-65536), jnp.float32)
    return jnp.where(parity == 1, hi, lo)


def _edge_mlp_body(xr, xc, pr, pc, l0, l1,
                   w1a, w1b, b1c, w20, w21, b20, b21, out):
    bf = jnp.bfloat16
    xrf = _unpack(xr[...], pr[...]).astype(bf)
    xcf = _unpack(xc[...], pc[...]).astype(bf)
    h = jnp.dot(xrf, w1a[...].astype(bf), preferred_element_type=jnp.float32)
    h += jnp.dot(xcf, w1b[...].astype(bf), preferred_element_type=jnp.float32)
    h = jnp.maximum(h + b1c[...], 0.0).astype(bf)
    m0 = jnp.dot(h[:, :256], w20[...].astype(bf),
                 preferred_element_type=jnp.float32)
    m0 = jnp.maximum(m0 + b20[...], 0.0)
    m1 = jnp.dot(h[:, 256:], w21[...].astype(bf),
                 preferred_element_type=jnp.float32)
    m1 = jnp.maximum(m1 + b21[...], 0.0)
    d = l0[...] - l1[...]
    p0 = 1.0 / (1.0 + jnp.exp(-BETA * d))
    msg = m0 * p0 + m1 * (1.0 - p0)
    out[0] = msg[:, :128]
    out[1] = msg[:, 128:]


def _edge_mlp(xr, xc, pr, pc, l0, l1, w1a, w1b, b1c, w20, w21, b20, b21):
    e = xr.shape[0]
    grid = (e // E_BLK,)
    blk = lambda r, c: pl.BlockSpec((r, c), lambda i: (i, 0))
    full = lambda r, c: pl.BlockSpec((r, c), lambda i: (0, 0))
    return pl.pallas_call(
        _edge_mlp_body,
        grid=grid,
        in_specs=[
            blk(E_BLK, 128), blk(E_BLK, 128),
            blk(E_BLK, 1), blk(E_BLK, 1), blk(E_BLK, 1), blk(E_BLK, 1),
            full(128, 512), full(128, 512), full(1, 512),
            full(256, 256), full(256, 256), full(1, 256), full(1, 256),
        ],
        out_specs=pl.BlockSpec((2, E_BLK, 128), lambda i: (0, i, 0)),
        out_shape=jax.ShapeDtypeStruct((2, e, 128), jnp.float32),
    )(xr, xc, pr, pc, l0, l1, w1a, w1b, b1c, w20, w21, b20, b21)


def _sc_gather_body(xpk_hbm, rowg_hbm, colg_hbm, xr_hbm, xc_hbm,
                    idx_r, idx_c, bufr, bufc, x_sp,
                    gsr0, gsr1, gsc0, gsc1, osr0, osr1, osc0, osc1):
    c = lax.axis_index("c")
    s = lax.axis_index("s")
    wid = s * 2 + c
    base = wid * (G_CHUNKS * CHUNK)
    @pl.when(s < 15)
    def _():
        pltpu.sync_copy(
            xpk_hbm.at[pl.ds(s * PK_ROWS_PER_TILE, PK_ROWS_PER_TILE), :],
            x_sp.at[pl.ds(s * PK_ROWS_PER_TILE, PK_ROWS_PER_TILE), :])

    @pl.when(s == 15)
    def _():
        pltpu.sync_copy(
            xpk_hbm.at[pl.ds(15 * PK_ROWS_PER_TILE, PK_LAST), :],
            x_sp.at[pl.ds(15 * PK_ROWS_PER_TILE, PK_LAST), :])

    pltpu.sync_copy(rowg_hbm.at[wid], idx_r)
    pltpu.sync_copy(colg_hbm.at[wid], idx_c)
    plsc.subcore_barrier()

    gsems_r = (gsr0, gsr1)
    gsems_c = (gsc0, gsc1)
    osems_r = (osr0, osr1)
    osems_c = (osc0, osc1)

    def out_r(j, p):
        return pltpu.make_async_copy(
            bufr.at[p], xr_hbm.at[pl.ds(base + j * CHUNK, CHUNK), :],
            osems_r[p])

    def out_c(j, p):
        return pltpu.make_async_copy(
            bufc.at[p], xc_hbm.at[pl.ds(base + j * CHUNK, CHUNK), :],
            osems_c[p])

    for p in (0, 1):
        pltpu.async_copy(x_sp.at[idx_r.at[p]], bufr.at[p], gsems_r[p])
        pltpu.async_copy(x_sp.at[idx_c.at[p]], bufc.at[p], gsems_c[p])

    def pair_body(g, carry):
        for p in (0, 1):
            j = 2 * g + p
            pltpu.make_async_copy(x_sp.at[idx_r.at[0]], bufr.at[p],
                                  gsems_r[p]).wait()
            out_r(j, p).start()
            pltpu.make_async_copy(x_sp.at[idx_c.at[0]], bufc.at[p],
                                  gsems_c[p]).wait()
            out_c(j, p).start()

            @pl.when(j + 2 < G_CHUNKS)
            def _():
                out_r(j, p).wait()
                pltpu.async_copy(x_sp.at[idx_r.at[j + 2]], bufr.at[p],
                                 gsems_r[p])
                out_c(j, p).wait()
                pltpu.async_copy(x_sp.at[idx_c.at[j + 2]], bufc.at[p],
                                 gsems_c[p])
        return carry

    lax.fori_loop(0, G_CHUNKS // 2, pair_body, 0, unroll=False)
    for p in (0, 1):
        out_r(0, p).wait()
        out_c(0, p).wait()


def _sc_gather(xpk, rowg, colg):
    mesh = plsc.VectorSubcoreMesh(core_axis_name="c", subcore_axis_name="s")
    f = pl.kernel(
        _sc_gather_body,
        out_type=(jax.ShapeDtypeStruct((EP, 128), jnp.int32),
                  jax.ShapeDtypeStruct((EP, 128), jnp.int32)),
        mesh=mesh,
        scratch_types=[
            pltpu.VMEM((G_CHUNKS, CHUNK), jnp.int32),
            pltpu.VMEM((G_CHUNKS, CHUNK), jnp.int32),
            pltpu.VMEM((2, CHUNK, 128), jnp.int32),
            pltpu.VMEM((2, CHUNK, 128), jnp.int32),
            pltpu.VMEM_SHARED((NH, 128), jnp.int32),
        ] + [pltpu.SemaphoreType.DMA] * 8,
    )
    return f(xpk, rowg, colg)


def _sc_scatter_body(msg_hbm, idx_hbm, zeros_hbm, out_hbm,
                     idx_v, buf, agg_sh, sem):
    c = lax.axis_index("c")
    s = lax.axis_index("s")
    pltpu.sync_copy(zeros_hbm.at[pl.ds(s * ROWS_PER_TILE, ROWS_PER_TILE), :],
                    agg_sh.at[pl.ds(s * ROWS_PER_TILE, ROWS_PER_TILE), :])
    pltpu.sync_copy(idx_hbm.at[s], idx_v)
    plsc.subcore_barrier()

    base = s * (CHUNKS_PER_TILE * CHUNK)

    def body(j, carry):
        pltpu.sync_copy(msg_hbm.at[c, pl.ds(base + j * CHUNK, CHUNK), :], buf)
        pltpu.sync_copy(buf, agg_sh.at[idx_v.at[j]], add=True)
        return carry

    lax.fori_loop(0, CHUNKS_PER_TILE, body, 0, unroll=False)
    plsc.subcore_barrier()

    @pl.when(s < 15)
    def _():
        pltpu.sync_copy(agg_sh.at[pl.ds(s * ROWS_PER_TILE, ROWS_PER_TILE), :],
                        out_hbm.at[c, pl.ds(s * ROWS_PER_TILE, ROWS_PER_TILE), :])

    @pl.when(s == 15)
    def _():
        pltpu.sync_copy(agg_sh.at[pl.ds(15 * ROWS_PER_TILE, N - 15 * ROWS_PER_TILE), :],
                        out_hbm.at[c, pl.ds(15 * ROWS_PER_TILE, N - 15 * ROWS_PER_TILE), :])


def _sc_scatter(msg2, idx3, zeros):
    mesh = plsc.VectorSubcoreMesh(core_axis_name="c", subcore_axis_name="s")
    f = pl.kernel(
        _sc_scatter_body,
        out_type=jax.ShapeDtypeStruct((2, N, 128), jnp.float32),
        mesh=mesh,
        scratch_types=[
            pltpu.VMEM((CHUNKS_PER_TILE, CHUNK), jnp.int32),
            pltpu.VMEM((CHUNK, 128), jnp.float32),
            pltpu.VMEM_SHARED((AGG_ROWS, 128), jnp.float32),
            pltpu.SemaphoreType.DMA,
        ],
    )
    return f(msg2, idx3, zeros)


def _node_mlp_body(x, agg0, agg1, wo1x, wo1a0, wo1a1, bo1, wo2, bo2, wo3, bo3, out):
    h1 = jnp.dot(x[...], wo1x[...], preferred_element_type=jnp.float32)
    h1 += jnp.dot(agg0[...], wo1a0[...], preferred_element_type=jnp.float32)
    h1 += jnp.dot(agg1[...], wo1a1[...], preferred_element_type=jnp.float32)
    h1 = jnp.maximum(h1 + bo1[...], 0.0)
    h2 = jnp.dot(h1, wo2[...], preferred_element_type=jnp.float32)
    h2 = jnp.maximum(h2 + bo2[...], 0.0)
    out[...] = x[...] + jnp.dot(h2, wo3[...], preferred_element_type=jnp.float32) + bo3[...]


def _node_mlp(x, agg0, agg1, wo1x, wo1a0, wo1a1, bo1, wo2, bo2, wo3, bo3):
    n = x.shape[0]
    grid = (n // N_BLK,)
    blk = lambda r, c: pl.BlockSpec((r, c), lambda i: (i, 0))
    full = lambda r, c: pl.BlockSpec((r, c), lambda i: (0, 0))
    return pl.pallas_call(
        _node_mlp_body,
        grid=grid,
        in_specs=[
            blk(N_BLK, 128), blk(N_BLK, 128), blk(N_BLK, 128),
            full(128, 256), full(128, 256), full(128, 256), full(1, 256),
            full(256, 256), full(1, 256),
            full(256, 128), full(1, 128),
        ],
        out_specs=blk(N_BLK, 128),
        out_shape=jax.ShapeDtypeStruct((n, 128), jnp.float32),
    )(x, agg0, agg1, wo1x, wo1a0, wo1a1, bo1, wo2, bo2, wo3, bo3)


def kernel(inputs, edge_index, logits, W1, b1, W2, b2, Wo1, bo1, Wo2, bo2, Wo3, bo3):
    row = edge_index[0]
    col = edge_index[1]

    w1a = jnp.concatenate([W1[0, :128, :], W1[1, :128, :]], axis=1)
    w1b = jnp.concatenate([W1[0, 128:, :], W1[1, 128:, :]], axis=1)
    b1c = jnp.concatenate([b1[0], b1[1]])[None, :]
    w20, w21 = W2[0], W2[1]
    b20, b21 = b2[0][None, :], b2[1][None, :]
    wo1x = Wo1[:128, :]
    wo1a0, wo1a1 = Wo1[128:256, :], Wo1[256:, :]
    bo1r, bo2r, bo3r = bo1[None, :], bo2[None, :], bo3[None, :]

    pad = EP - E
    l0 = jnp.pad(logits[0], (0, pad))[:, None]
    l1 = jnp.pad(logits[1], (0, pad))[:, None]
    rp = jnp.pad(row, (0, pad))
    cp = jnp.pad(col, (0, pad))
    prf = (rp >= NH).astype(jnp.int32)
    pcf = (cp >= NH).astype(jnp.int32)
    row_g = (rp - NH * prf).reshape(32, G_CHUNKS, CHUNK)
    col_g = (cp - NH * pcf).reshape(32, G_CHUNKS, CHUNK)
    pr = prf[:, None]
    pc = pcf[:, None]
    idx3 = jnp.pad(row, (0, pad), constant_values=N).reshape(16, CHUNKS_PER_TILE, CHUNK)
    zeros = jnp.zeros((AGG_ROWS, 128), jnp.float32)

    def step(x):
        lo = lax.bitcast_convert_type(
            x[:NH].astype(jnp.bfloat16), jnp.uint16).astype(jnp.int32)
        hi = lax.bitcast_convert_type(
            x[NH:].astype(jnp.bfloat16), jnp.uint16).astype(jnp.int32)
        xpk = jnp.bitwise_or(jnp.left_shift(hi, 16), lo)
        xr, xc = _sc_gather(xpk, row_g, col_g)
        msg2 = _edge_mlp(xr, xc, pr, pc, l0, l1, w1a, w1b,
                         b1c, w20, w21, b20, b21)
        agg2 = _sc_scatter(msg2, idx3, zeros)
        return _node_mlp(x, agg2[0], agg2[1], wo1x, wo1a0, wo1a1, bo1r,
                         Wo2, bo2r, Wo3, bo3r)

    x = inputs[0, :, :, 0]
    x1 = step(x)
    x2 = step(x1)
    return jnp.stack([x1, x2], axis=-1)[None]


# speedup vs baseline: 2.1555x; 1.1629x over previous
import functools

import jax
import jax.numpy as jnp
from jax import lax
from jax.experimental import pallas as pl
from jax.experimental.pallas import tpu as pltpu
from jax.experimental.pallas import tpu_sc as plsc

BETA = 0.5

N = 10000
NH = N // 2
E = 160000
EP = 163840
E_BLK = 2048
N_BLK = 1000
AGG_ROWS = 10112
ROWS_PER_TILE = 632
CHUNK = 128
CHUNKS_PER_TILE = EP // 16 // CHUNK
G_CHUNKS = EP // 32 // CHUNK
PK_ROWS_PER_TILE = 320
PK_LAST = NH - 15 * PK_ROWS_PER_TILE


def _unpack(words, parity):
    lo = lax.bitcast_convert_type(words << 16, jnp.float32)
    hi = lax.bitcast_convert_type(words & jnp.int32(-65536), jnp.float32)
    return jnp.where(parity == 1, hi, lo)


def _edge_mlp_body(xr, xc, dm,
                   w1a, w1b, b1c, w20, w21, b20, b21, out):
    bits = lax.bitcast_convert_type(dm[...], jnp.int32)
    pr = bits & 1
    pc = (bits >> 1) & 1
    d = lax.bitcast_convert_type(bits & jnp.int32(-4), jnp.float32)
    bf = jnp.bfloat16
    xrf = _unpack(xr[...], pr).astype(bf)
    xcf = _unpack(xc[...], pc).astype(bf)
    h = jnp.dot(xrf, w1a[...].astype(bf), preferred_element_type=jnp.float32)
    h += jnp.dot(xcf, w1b[...].astype(bf), preferred_element_type=jnp.float32)
    h = jnp.maximum(h + b1c[...], 0.0).astype(bf)
    m0 = jnp.dot(h[:, :256], w20[...].astype(bf),
                 preferred_element_type=jnp.float32)
    m0 = jnp.maximum(m0 + b20[...], 0.0)
    m1 = jnp.dot(h[:, 256:], w21[...].astype(bf),
                 preferred_element_type=jnp.float32)
    m1 = jnp.maximum(m1 + b21[...], 0.0)
    p0 = 1.0 / (1.0 + jnp.exp(-BETA * d))
    msg = m0 * p0 + m1 * (1.0 - p0)
    out[0] = msg[:, :128]
    out[1] = msg[:, 128:]


def _edge_mlp(xr, xc, dm, w1a, w1b, b1c, w20, w21, b20, b21):
    e = xr.shape[0]
    grid = (e // E_BLK,)
    blk = lambda r, c: pl.BlockSpec((r, c), lambda i: (i, 0))
    full = lambda r, c: pl.BlockSpec((r, c), lambda i: (0, 0))
    return pl.pallas_call(
        _edge_mlp_body,
        grid=grid,
        in_specs=[
            blk(E_BLK, 128), blk(E_BLK, 128), blk(E_BLK, 1),
            full(128, 512), full(128, 512), full(1, 512),
            full(256, 256), full(256, 256), full(1, 256), full(1, 256),
        ],
        out_specs=pl.BlockSpec((2, E_BLK, 128), lambda i: (0, i, 0)),
        out_shape=jax.ShapeDtypeStruct((2, e, 128), jnp.float32),
    )(xr, xc, dm, w1a, w1b, b1c, w20, w21, b20, b21)


def _sc_gather_body(xpk_hbm, rowg_hbm, colg_hbm, xr_hbm, xc_hbm,
                    idx_r, idx_c, bufr, bufc, x_sp,
                    gsr0, gsr1, gsc0, gsc1, osr0, osr1, osc0, osc1):
    c = lax.axis_index("c")
    s = lax.axis_index("s")
    wid = s * 2 + c
    base = wid * (G_CHUNKS * CHUNK)
    @pl.when(s < 15)
    def _():
        pltpu.sync_copy(
            xpk_hbm.at[pl.ds(s * PK_ROWS_PER_TILE, PK_ROWS_PER_TILE), :],
            x_sp.at[pl.ds(s * PK_ROWS_PER_TILE, PK_ROWS_PER_TILE), :])

    @pl.when(s == 15)
    def _():
        pltpu.sync_copy(
            xpk_hbm.at[pl.ds(15 * PK_ROWS_PER_TILE, PK_LAST), :],
            x_sp.at[pl.ds(15 * PK_ROWS_PER_TILE, PK_LAST), :])

    pltpu.sync_copy(rowg_hbm.at[wid], idx_r)
    pltpu.sync_copy(colg_hbm.at[wid], idx_c)
    plsc.subcore_barrier()

    gsems_r = (gsr0, gsr1)
    gsems_c = (gsc0, gsc1)
    osems_r = (osr0, osr1)
    osems_c = (osc0, osc1)

    def out_r(j, p):
        return pltpu.make_async_copy(
            bufr.at[p], xr_hbm.at[pl.ds(base + j * CHUNK, CHUNK), :],
            osems_r[p])

    def out_c(j, p):
        return pltpu.make_async_copy(
            bufc.at[p], xc_hbm.at[pl.ds(base + j * CHUNK, CHUNK), :],
            osems_c[p])

    for p in (0, 1):
        pltpu.async_copy(x_sp.at[idx_r.at[p]], bufr.at[p], gsems_r[p])
        pltpu.async_copy(x_sp.at[idx_c.at[p]], bufc.at[p], gsems_c[p])

    def pair_body(g, carry):
        for p in (0, 1):
            j = 2 * g + p
            pltpu.make_async_copy(x_sp.at[idx_r.at[0]], bufr.at[p],
                                  gsems_r[p]).wait()
            out_r(j, p).start()
            pltpu.make_async_copy(x_sp.at[idx_c.at[0]], bufc.at[p],
                                  gsems_c[p]).wait()
            out_c(j, p).start()

            @pl.when(j + 2 < G_CHUNKS)
            def _():
                out_r(j, p).wait()
                pltpu.async_copy(x_sp.at[idx_r.at[j + 2]], bufr.at[p],
                                 gsems_r[p])
                out_c(j, p).wait()
                pltpu.async_copy(x_sp.at[idx_c.at[j + 2]], bufc.at[p],
                                 gsems_c[p])
        return carry

    lax.fori_loop(0, G_CHUNKS // 2, pair_body, 0, unroll=False)
    for p in (0, 1):
        out_r(0, p).wait()
        out_c(0, p).wait()


def _sc_gather(xpk, rowg, colg):
    mesh = plsc.VectorSubcoreMesh(core_axis_name="c", subcore_axis_name="s")
    f = pl.kernel(
        _sc_gather_body,
        out_type=(jax.ShapeDtypeStruct((EP, 128), jnp.int32),
                  jax.ShapeDtypeStruct((EP, 128), jnp.int32)),
        mesh=mesh,
        scratch_types=[
            pltpu.VMEM((G_CHUNKS, CHUNK), jnp.int32),
            pltpu.VMEM((G_CHUNKS, CHUNK), jnp.int32),
            pltpu.VMEM((2, CHUNK, 128), jnp.int32),
            pltpu.VMEM((2, CHUNK, 128), jnp.int32),
            pltpu.VMEM_SHARED((NH, 128), jnp.int32),
        ] + [pltpu.SemaphoreType.DMA] * 8,
    )
    return f(xpk, rowg, colg)


def _sc_scatter_body(msg_hbm, idx_hbm, zeros_hbm, out_hbm,
                     idx_v, buf, agg_sh, sem):
    c = lax.axis_index("c")
    s = lax.axis_index("s")
    pltpu.sync_copy(zeros_hbm.at[pl.ds(s * ROWS_PER_TILE, ROWS_PER_TILE), :],
                    agg_sh.at[pl.ds(s * ROWS_PER_TILE, ROWS_PER_TILE), :])
    pltpu.sync_copy(idx_hbm.at[s], idx_v)
    plsc.subcore_barrier()

    base = s * (CHUNKS_PER_TILE * CHUNK)

    def body(j, carry):
        pltpu.sync_copy(msg_hbm.at[c, pl.ds(base + j * CHUNK, CHUNK), :], buf)
        pltpu.sync_copy(buf, agg_sh.at[idx_v.at[j]], add=True)
        return carry

    lax.fori_loop(0, CHUNKS_PER_TILE, body, 0, unroll=False)
    plsc.subcore_barrier()

    @pl.when(s < 15)
    def _():
        pltpu.sync_copy(agg_sh.at[pl.ds(s * ROWS_PER_TILE, ROWS_PER_TILE), :],
                        out_hbm.at[c, pl.ds(s * ROWS_PER_TILE, ROWS_PER_TILE), :])

    @pl.when(s == 15)
    def _():
        pltpu.sync_copy(agg_sh.at[pl.ds(15 * ROWS_PER_TILE, N - 15 * ROWS_PER_TILE), :],
                        out_hbm.at[c, pl.ds(15 * ROWS_PER_TILE, N - 15 * ROWS_PER_TILE), :])


def _sc_scatter(msg2, idx3, zeros):
    mesh = plsc.VectorSubcoreMesh(core_axis_name="c", subcore_axis_name="s")
    f = pl.kernel(
        _sc_scatter_body,
        out_type=jax.ShapeDtypeStruct((2, N, 128), jnp.float32),
        mesh=mesh,
        scratch_types=[
            pltpu.VMEM((CHUNKS_PER_TILE, CHUNK), jnp.int32),
            pltpu.VMEM((CHUNK, 128), jnp.float32),
            pltpu.VMEM_SHARED((AGG_ROWS, 128), jnp.float32),
            pltpu.SemaphoreType.DMA,
        ],
    )
    return f(msg2, idx3, zeros)


def _node_mlp_body(x, agg0, agg1, wo1x, wo1a0, wo1a1, bo1, wo2, bo2, wo3, bo3, out):
    h1 = jnp.dot(x[...], wo1x[...], preferred_element_type=jnp.float32)
    h1 += jnp.dot(agg0[...], wo1a0[...], preferred_element_type=jnp.float32)
    h1 += jnp.dot(agg1[...], wo1a1[...], preferred_element_type=jnp.float32)
    h1 = jnp.maximum(h1 + bo1[...], 0.0)
    h2 = jnp.dot(h1, wo2[...], preferred_element_type=jnp.float32)
    h2 = jnp.maximum(h2 + bo2[...], 0.0)
    out[...] = x[...] + jnp.dot(h2, wo3[...], preferred_element_type=jnp.float32) + bo3[...]


def _node_mlp(x, agg0, agg1, wo1x, wo1a0, wo1a1, bo1, wo2, bo2, wo3, bo3):
    n = x.shape[0]
    grid = (n // N_BLK,)
    blk = lambda r, c: pl.BlockSpec((r, c), lambda i: (i, 0))
    full = lambda r, c: pl.BlockSpec((r, c), lambda i: (0, 0))
    return pl.pallas_call(
        _node_mlp_body,
        grid=grid,
        in_specs=[
            blk(N_BLK, 128), blk(N_BLK, 128), blk(N_BLK, 128),
            full(128, 256), full(128, 256), full(128, 256), full(1, 256),
            full(256, 256), full(1, 256),
            full(256, 128), full(1, 128),
        ],
        out_specs=blk(N_BLK, 128),
        out_shape=jax.ShapeDtypeStruct((n, 128), jnp.float32),
    )(x, agg0, agg1, wo1x, wo1a0, wo1a1, bo1, wo2, bo2, wo3, bo3)


def kernel(inputs, edge_index, logits, W1, b1, W2, b2, Wo1, bo1, Wo2, bo2, Wo3, bo3):
    row = edge_index[0]
    col = edge_index[1]

    w1a = jnp.concatenate([W1[0, :128, :], W1[1, :128, :]], axis=1)
    w1b = jnp.concatenate([W1[0, 128:, :], W1[1, 128:, :]], axis=1)
    b1c = jnp.concatenate([b1[0], b1[1]])[None, :]
    w20, w21 = W2[0], W2[1]
    b20, b21 = b2[0][None, :], b2[1][None, :]
    wo1x = Wo1[:128, :]
    wo1a0, wo1a1 = Wo1[128:256, :], Wo1[256:, :]
    bo1r, bo2r, bo3r = bo1[None, :], bo2[None, :], bo3[None, :]

    pad = EP - E
    rp = jnp.pad(row, (0, pad))
    cp = jnp.pad(col, (0, pad))
    prf = (rp >= NH).astype(jnp.int32)
    pcf = (cp >= NH).astype(jnp.int32)
    row_g = (rp - NH * prf).reshape(32, G_CHUNKS, CHUNK)
    col_g = (cp - NH * pcf).reshape(32, G_CHUNKS, CHUNK)
    d = jnp.pad(logits[0] - logits[1], (0, pad))
    dbits = lax.bitcast_convert_type(d, jnp.int32)
    dm = lax.bitcast_convert_type(
        (dbits & jnp.int32(-4)) | prf | (pcf << 1), jnp.float32)[:, None]
    idx3 = jnp.pad(row, (0, pad), constant_values=N).reshape(16, CHUNKS_PER_TILE, CHUNK)
    zeros = jnp.zeros((AGG_ROWS, 128), jnp.float32)

    def step(x):
        lo = lax.bitcast_convert_type(
            x[:NH].astype(jnp.bfloat16), jnp.uint16).astype(jnp.int32)
        hi = lax.bitcast_convert_type(
            x[NH:].astype(jnp.bfloat16), jnp.uint16).astype(jnp.int32)
        xpk = jnp.bitwise_or(jnp.left_shift(hi, 16), lo)
        xr, xc = _sc_gather(xpk, row_g, col_g)
        msg2 = _edge_mlp(xr, xc, dm, w1a, w1b, b1c, w20, w21, b20, b21)
        agg2 = _sc_scatter(msg2, idx3, zeros)
        return _node_mlp(x, agg2[0], agg2[1], wo1x, wo1a0, wo1a1, bo1r,
                         Wo2, bo2r, Wo3, bo3r)

    x = inputs[0, :, :, 0]
    x1 = step(x)
    x2 = step(x1)
    return jnp.stack([x1, x2], axis=-1)[None]
